# baseline (speedup 1.0000x reference)
import functools

import jax
import jax.numpy as jnp
from jax import lax
from jax.experimental import pallas as pl
from jax.experimental.pallas import tpu as pltpu
from jax.experimental.pallas import tpu_sc as plsc

ROWS = 128
COLS = 32768
LANES = 16
NV = COLS // LANES
NWORKERS = 32
ROWS_PER_W = ROWS // NWORKERS
RADIX = 2048
SHIFTS = (0, 11, 22)
MASKS = (2047, 2047, 1023)
NPASSES = 3
K = 8
SV = NV // K
CH = COLS // K
CHB = 12
RB = 11

_SIGN = jnp.int32(-2147483648)


def _to_sortable(k):
    return k ^ ((k >> 31) | _SIGN)


def _from_sortable(k):
    return k ^ ((~(k >> 31)) | _SIGN)


def _bcast_last(v):
    idx = jnp.full((LANES,), LANES - 1, jnp.int32)
    return v[idx]


def _sort_body(
    x_hbm, out_hbm, buf_a, buf_b, o0, o1, o2, o3, o4, o5, o6, o7, histn,
    sem_in, sem_out,
):
    cid = lax.axis_index("c")
    sid = lax.axis_index("s")
    wid = sid * 2 + cid
    offs = (o0, o1, o2, o3, o4, o5, o6, o7)

    ones = jnp.ones((LANES,), jnp.int32)
    zeros = jnp.zeros((LANES,), jnp.int32)

    @plsc.parallel_loop(0, K * RADIX // LANES // 4, unroll=2)
    def zero_step(h):
        for j in range(4):
            histn[pl.ds((h * 4 + j) * LANES, LANES)] = zeros

    def do_row(r, out_copy):
        row = wid * ROWS_PER_W + r
        in_copy = pltpu.async_copy(x_hbm.at[row], buf_a, sem_in)

        in_copy.wait()

        @plsc.parallel_loop(0, SV, unroll=2)
        def h0_step(i):
            for k in range(K):
                kv = _to_sortable(buf_a[pl.ds(i * LANES + k * CH, LANES)])
                d = kv & MASKS[0]
                plsc.addupdate_scatter(histn, (d + k * RADIX,), ones)

        for p in range(NPASSES):
            src, dst = (buf_a, buf_b) if p % 2 == 0 else (buf_b, buf_a)
            shift = SHIFTS[p]
            mask = MASKS[p]

            @plsc.parallel_loop(0, (mask + 1) // LANES, unroll=2, carry=zeros)
            def mk_offs(h, basev):
                t = [histn[pl.ds(h * LANES + k * RADIX, LANES)] for k in range(K)]
                tot = ((t[0] + t[1]) + (t[2] + t[3])) + ((t[4] + t[5]) + (t[6] + t[7]))
                inc = plsc.cumsum(tot)
                run = inc - tot + (basev - 1)
                for k in range(K):
                    offs[k][pl.ds(h * LANES, LANES)] = run
                    run = run + t[k]
                    histn[pl.ds(h * LANES + k * RADIX, LANES)] = zeros
                return basev + _bcast_last(inc)

            def perm_step(i, c, _src=src, _dst=dst, _shift=shift, _mask=mask, _p=p):
                kvs, dss, poss = [], [], []
                for k in range(K):
                    kv = _src[pl.ds(i * LANES + k * CH, LANES)]
                    if _p == 0:
                        kv = _to_sortable(kv)
                    d = (kv >> _shift) & _mask
                    occ, _ = plsc.scan_count(d)
                    pos = plsc.load_gather(offs[k], (d,)) + occ
                    kvs.append(kv)
                    dss.append(d)
                    poss.append(pos)
                for k in range(K):
                    out_v = kvs[k]
                    if _p == NPASSES - 1:
                        out_v = _from_sortable(out_v)
                    plsc.store_scatter(_dst, (poss[k],), out_v)
                    plsc.addupdate_scatter(offs[k], (dss[k],), ones)
                    if _p < NPASSES - 1:
                        dn = (kvs[k] >> SHIFTS[_p + 1]) & MASKS[_p + 1]
                        hidx = ((poss[k] >> CHB) << RB) + dn
                        plsc.addupdate_scatter(histn, (hidx,), ones)
                return c

            if p == 0 and out_copy is not None:
                out_copy.wait()
            lax.fori_loop(0, SV, perm_step, 0)

        return pltpu.async_copy(buf_b, out_hbm.at[row], sem_out)

    out_copy = None
    for r in range(ROWS_PER_W):
        out_copy = do_row(r, out_copy)
    out_copy.wait()


@jax.jit
def kernel(x):
    xi = lax.bitcast_convert_type(x, jnp.int32)
    run = pl.kernel(
        _sort_body,
        out_type=jax.ShapeDtypeStruct((ROWS, COLS), jnp.int32),
        mesh=plsc.VectorSubcoreMesh(core_axis_name="c", subcore_axis_name="s"),
        compiler_params=pltpu.CompilerParams(needs_layout_passes=False),
        scratch_types=[
            pltpu.VMEM((COLS,), jnp.int32),
            pltpu.VMEM((COLS,), jnp.int32),
            pltpu.VMEM((RADIX,), jnp.int32),
            pltpu.VMEM((RADIX,), jnp.int32),
            pltpu.VMEM((RADIX,), jnp.int32),
            pltpu.VMEM((RADIX,), jnp.int32),
            pltpu.VMEM((RADIX,), jnp.int32),
            pltpu.VMEM((RADIX,), jnp.int32),
            pltpu.VMEM((RADIX,), jnp.int32),
            pltpu.VMEM((RADIX,), jnp.int32),
            pltpu.VMEM((K * RADIX,), jnp.int32),
            pltpu.SemaphoreType.DMA,
            pltpu.SemaphoreType.DMA,
        ],
    )
    return lax.bitcast_convert_type(run(xi), jnp.float32)


# speedup vs baseline: 1.0083x; 1.0083x over previous
import functools

import jax
import jax.numpy as jnp
from jax import lax
from jax.experimental import pallas as pl
from jax.experimental.pallas import tpu as pltpu
from jax.experimental.pallas import tpu_sc as plsc

ROWS = 128
COLS = 32768
LANES = 16
NV = COLS // LANES
NWORKERS = 32
ROWS_PER_W = ROWS // NWORKERS
RADIX = 2048
SHIFTS = (0, 11, 22)
MASKS = (2047, 2047, 1023)
NPASSES = 3
K = 8
SV = NV // K
CH = COLS // K
CHB = 12
RB = 11

_SIGN = jnp.int32(-2147483648)


def _to_sortable(k):
    return k ^ ((k >> 31) | _SIGN)


def _from_sortable(k):
    return k ^ ((~(k >> 31)) | _SIGN)


def _bcast_last(v):
    idx = jnp.full((LANES,), LANES - 1, jnp.int32)
    return v[idx]


def _sort_body(x_hbm, out_hbm, buf_a, buf_b, o0, o1, o2, o3, o4, o5, o6, o7, histn):
    cid = lax.axis_index("c")
    sid = lax.axis_index("s")
    wid = sid * 2 + cid
    offs = (o0, o1, o2, o3, o4, o5, o6, o7)

    ones = jnp.ones((LANES,), jnp.int32)
    zeros = jnp.zeros((LANES,), jnp.int32)

    @plsc.parallel_loop(0, K * RADIX // LANES // 4, unroll=2)
    def zero_step(h):
        for j in range(4):
            histn[pl.ds((h * 4 + j) * LANES, LANES)] = zeros

    def do_row(r, carry):
        row = wid * ROWS_PER_W + r
        pltpu.sync_copy(x_hbm.at[row], buf_a)

        @plsc.parallel_loop(0, SV, unroll=4)
        def h0_step(i):
            for k in range(K):
                kv = _to_sortable(buf_a[pl.ds(i * LANES + k * CH, LANES)])
                d = kv & MASKS[0]
                plsc.addupdate_scatter(histn, (d + k * RADIX,), ones)

        for p in range(NPASSES):
            src, dst = (buf_a, buf_b) if p % 2 == 0 else (buf_b, buf_a)
            shift = SHIFTS[p]
            mask = MASKS[p]

            @plsc.parallel_loop(0, (mask + 1) // LANES, unroll=4, carry=zeros)
            def mk_offs(h, basev):
                t = [histn[pl.ds(h * LANES + k * RADIX, LANES)] for k in range(K)]
                tot = ((t[0] + t[1]) + (t[2] + t[3])) + ((t[4] + t[5]) + (t[6] + t[7]))
                inc = plsc.cumsum(tot)
                run = inc - tot + (basev - 1)
                for k in range(K):
                    offs[k][pl.ds(h * LANES, LANES)] = run
                    run = run + t[k]
                    histn[pl.ds(h * LANES + k * RADIX, LANES)] = zeros
                return basev + _bcast_last(inc)

            def perm_step(i, c, _src=src, _dst=dst, _shift=shift, _mask=mask, _p=p):
                kvs, dss, poss = [], [], []
                for k in range(K):
                    kv = _src[pl.ds(i * LANES + k * CH, LANES)]
                    if _p == 0:
                        kv = _to_sortable(kv)
                    d = (kv >> _shift) & _mask
                    occ, _ = plsc.scan_count(d)
                    pos = plsc.load_gather(offs[k], (d,)) + occ
                    kvs.append(kv)
                    dss.append(d)
                    poss.append(pos)
                for k in range(K):
                    out_v = kvs[k]
                    if _p == NPASSES - 1:
                        out_v = _from_sortable(out_v)
                    plsc.store_scatter(_dst, (poss[k],), out_v)
                    plsc.addupdate_scatter(offs[k], (dss[k],), ones)
                    if _p < NPASSES - 1:
                        dn = (kvs[k] >> SHIFTS[_p + 1]) & MASKS[_p + 1]
                        hidx = ((poss[k] >> CHB) << RB) + dn
                        plsc.addupdate_scatter(histn, (hidx,), ones)
                return c

            lax.fori_loop(0, SV, perm_step, 0)

        pltpu.sync_copy(buf_b, out_hbm.at[row])
        return carry

    lax.fori_loop(0, ROWS_PER_W, do_row, 0)


@jax.jit
def kernel(x):
    xi = lax.bitcast_convert_type(x, jnp.int32)
    run = pl.kernel(
        _sort_body,
        out_type=jax.ShapeDtypeStruct((ROWS, COLS), jnp.int32),
        mesh=plsc.VectorSubcoreMesh(core_axis_name="c", subcore_axis_name="s"),
        compiler_params=pltpu.CompilerParams(needs_layout_passes=False),
        scratch_types=[
            pltpu.VMEM((COLS,), jnp.int32),
            pltpu.VMEM((COLS,), jnp.int32),
            pltpu.VMEM((RADIX,), jnp.int32),
            pltpu.VMEM((RADIX,), jnp.int32),
            pltpu.VMEM((RADIX,), jnp.int32),
            pltpu.VMEM((RADIX,), jnp.int32),
            pltpu.VMEM((RADIX,), jnp.int32),
            pltpu.VMEM((RADIX,), jnp.int32),
            pltpu.VMEM((RADIX,), jnp.int32),
            pltpu.VMEM((RADIX,), jnp.int32),
            pltpu.VMEM((K * RADIX,), jnp.int32),
        ],
    )
    return lax.bitcast_convert_type(run(xi), jnp.float32)


# speedup vs baseline: 1.0112x; 1.0028x over previous
import jax
import jax.numpy as jnp
from jax import lax
from jax.experimental import pallas as pl
from jax.experimental.pallas import tpu as pltpu
from jax.experimental.pallas import tpu_sc as plsc

ROWS = 128
COLS = 32768
LANES = 16
NV = COLS // LANES
NWORKERS = 32
ROWS_PER_W = ROWS // NWORKERS
RADIX = 2048
SHIFTS = (0, 11, 22)
MASKS = (2047, 2047, 1023)
NPASSES = 3
K = 8
SV = NV // K
CH = COLS // K
CHB = 12
RB = 11

_SIGN = jnp.int32(-2147483648)


def _to_sortable(k):
    return k ^ ((k >> 31) | _SIGN)


def _from_sortable(k):
    return k ^ ((~(k >> 31)) | _SIGN)


def _bcast_last(v):
    idx = jnp.full((LANES,), LANES - 1, jnp.int32)
    return v[idx]


def _sort_body(x_hbm, out_hbm, buf_a, buf_b, o0, o1, o2, o3, o4, o5, o6, o7, histn):
    cid = lax.axis_index("c")
    sid = lax.axis_index("s")
    wid = sid * 2 + cid
    offs = (o0, o1, o2, o3, o4, o5, o6, o7)

    ones = jnp.ones((LANES,), jnp.int32)
    zeros = jnp.zeros((LANES,), jnp.int32)

    @plsc.parallel_loop(0, K * RADIX // LANES // 4, unroll=2)
    def zero_step(h):
        for j in range(4):
            histn[pl.ds((h * 4 + j) * LANES, LANES)] = zeros

    def do_row(r, carry):
        row = wid * ROWS_PER_W + r
        pltpu.sync_copy(x_hbm.at[row], buf_a)

        @plsc.parallel_loop(0, SV, unroll=2)
        def h0_step(i):
            for k in range(K):
                kv = _to_sortable(buf_a[pl.ds(i * LANES + k * CH, LANES)])
                d = kv & MASKS[0]
                plsc.addupdate_scatter(histn, (d + k * RADIX,), ones)

        for p in range(NPASSES):
            src, dst = (buf_a, buf_b) if p % 2 == 0 else (buf_b, buf_a)
            shift = SHIFTS[p]
            mask = MASKS[p]

            @plsc.parallel_loop(0, (mask + 1) // LANES, unroll=2, carry=zeros)
            def mk_offs(h, basev):
                t = [histn[pl.ds(h * LANES + k * RADIX, LANES)] for k in range(K)]
                tot = ((t[0] + t[1]) + (t[2] + t[3])) + ((t[4] + t[5]) + (t[6] + t[7]))
                inc = plsc.cumsum(tot)
                run = inc - tot + (basev - 1)
                for k in range(K):
                    offs[k][pl.ds(h * LANES, LANES)] = run
                    run = run + t[k]
                    histn[pl.ds(h * LANES + k * RADIX, LANES)] = zeros
                return basev + _bcast_last(inc)

            def perm_step(i, c, _src=src, _dst=dst, _shift=shift, _mask=mask, _p=p):
                kvs, dss, poss = [], [], []
                for k in range(K):
                    kv = _src[pl.ds(i * LANES + k * CH, LANES)]
                    if _p == 0:
                        kv = _to_sortable(kv)
                    d = (kv >> _shift) & _mask
                    occ, _ = plsc.scan_count(d)
                    pos = plsc.load_gather(offs[k], (d,)) + occ
                    kvs.append(kv)
                    dss.append(d)
                    poss.append(pos)
                for k in range(K):
                    out_v = kvs[k]
                    if _p == NPASSES - 1:
                        out_v = _from_sortable(out_v)
                    plsc.store_scatter(_dst, (poss[k],), out_v)
                    plsc.addupdate_scatter(offs[k], (dss[k],), ones)
                    if _p < NPASSES - 1:
                        dn = (kvs[k] >> SHIFTS[_p + 1]) & MASKS[_p + 1]
                        hidx = ((poss[k] >> CHB) << RB) + dn
                        plsc.addupdate_scatter(histn, (hidx,), ones)
                return c

            lax.fori_loop(0, SV, perm_step, 0)

        pltpu.sync_copy(buf_b, out_hbm.at[row])
        return carry

    lax.fori_loop(0, ROWS_PER_W, do_row, 0)


@jax.jit
def kernel(x):
    xi = lax.bitcast_convert_type(x, jnp.int32)
    run = pl.kernel(
        _sort_body,
        out_type=jax.ShapeDtypeStruct((ROWS, COLS), jnp.int32),
        mesh=plsc.VectorSubcoreMesh(core_axis_name="c", subcore_axis_name="s"),
        compiler_params=pltpu.CompilerParams(needs_layout_passes=False),
        scratch_types=[
            pltpu.VMEM((COLS,), jnp.int32),
            pltpu.VMEM((COLS,), jnp.int32),
            pltpu.VMEM((RADIX,), jnp.int32),
            pltpu.VMEM((RADIX,), jnp.int32),
            pltpu.VMEM((RADIX,), jnp.int32),
            pltpu.VMEM((RADIX,), jnp.int32),
            pltpu.VMEM((RADIX,), jnp.int32),
            pltpu.VMEM((RADIX,), jnp.int32),
            pltpu.VMEM((RADIX,), jnp.int32),
            pltpu.VMEM((RADIX,), jnp.int32),
            pltpu.VMEM((K * RADIX,), jnp.int32),
        ],
    )
    return lax.bitcast_convert_type(run(xi), jnp.float32)
